# baseline (speedup 1.0000x reference)
import functools

import jax
import jax.numpy as jnp
from jax import lax
from jax.experimental import pallas as pl
from jax.experimental.pallas import tpu as pltpu
from jax.experimental.pallas import tpu_sc as plsc

NUM_CORES = 2
NUM_SUBCORES = 16
NUM_WORKERS = NUM_CORES * NUM_SUBCORES
LANES = 16

BATCH = 4096
SEQ = 200
DIM = 64
CHUNK = 40
CHUNKS_PER_ELEM = SEQ // CHUNK
PER_W = BATCH // NUM_WORKERS
GROUP = 8
VREGS = DIM // LANES


def _make_kernel():
    mesh = plsc.VectorSubcoreMesh(core_axis_name="c", subcore_axis_name="s")

    @functools.partial(
        pl.kernel,
        mesh=mesh,
        compiler_params=pltpu.CompilerParams(use_tc_tiling_on_sc=False),
        out_type=jax.ShapeDtypeStruct((BATCH, DIM), jnp.float32),
        scratch_types=[
            pltpu.VMEM((GROUP * CHUNKS_PER_ELEM, CHUNK), jnp.int32),
            pltpu.VMEM((SEQ, DIM), jnp.float32),
            pltpu.VMEM((GROUP, DIM), jnp.float32),
            pltpu.SemaphoreType.DMA,
        ],
    )
    def label_mean(labels_hbm, table_hbm, out_hbm, idx_v, rows_v, out_v, sem):
        wid = lax.axis_index("s") * NUM_CORES + lax.axis_index("c")
        base = wid * PER_W

        def group_body(g, carry):
            b0 = base + g * GROUP
            pltpu.sync_copy(
                labels_hbm.at[
                    pl.ds(b0 * CHUNKS_PER_ELEM, GROUP * CHUNKS_PER_ELEM)
                ],
                idx_v,
            )
            scale = jnp.float32(1.0 / SEQ)
            for e in range(GROUP):
                copies = [
                    pltpu.async_copy(
                        table_hbm.at[idx_v.at[e * CHUNKS_PER_ELEM + c]],
                        rows_v.at[pl.ds(c * CHUNK, CHUNK)],
                        sem,
                    )
                    for c in range(CHUNKS_PER_ELEM)
                ]
                for cp in copies:
                    cp.wait()

                def red_body(r, accs):
                    return tuple(
                        accs[k] + rows_v[r, pl.ds(k * LANES, LANES)]
                        for k in range(VREGS)
                    )

                zero = jnp.zeros((LANES,), jnp.float32)
                accs = lax.fori_loop(0, SEQ, red_body, (zero,) * VREGS)
                for k in range(VREGS):
                    out_v[e, pl.ds(k * LANES, LANES)] = accs[k] * scale
            pltpu.sync_copy(out_v, out_hbm.at[pl.ds(b0, GROUP)])
            return carry

        lax.fori_loop(0, PER_W // GROUP, group_body, 0)

    return label_mean


_label_mean = _make_kernel()


@jax.jit
def kernel(labels, table):
    labels2d = labels.astype(jnp.int32).reshape(
        BATCH * CHUNKS_PER_ELEM, CHUNK
    )
    return _label_mean(labels2d, table)


# speedup vs baseline: 1.1860x; 1.1860x over previous
import functools

import jax
import jax.numpy as jnp
from jax import lax
from jax.experimental import pallas as pl
from jax.experimental.pallas import tpu as pltpu
from jax.experimental.pallas import tpu_sc as plsc

NUM_CORES = 2
NUM_SUBCORES = 16
NUM_WORKERS = NUM_CORES * NUM_SUBCORES
LANES = 16

BATCH = 4096
SEQ = 200
DIM = 64
PER_W = BATCH // NUM_WORKERS
GROUP = 8
C0, C1 = 128, SEQ - 128
VREGS = DIM // LANES
UNROLL = 8


def _make_kernel():
    mesh = plsc.VectorSubcoreMesh(core_axis_name="c", subcore_axis_name="s")

    @functools.partial(
        pl.kernel,
        mesh=mesh,
        compiler_params=pltpu.CompilerParams(use_tc_tiling_on_sc=False),
        out_type=jax.ShapeDtypeStruct((BATCH, DIM), jnp.float32),
        scratch_types=[
            pltpu.VMEM((PER_W * SEQ,), jnp.int32),
            pltpu.VMEM((SEQ, DIM), jnp.float32),
            pltpu.VMEM((SEQ, DIM), jnp.float32),
            pltpu.VMEM((GROUP, DIM), jnp.float32),
            pltpu.SemaphoreType.DMA,
            pltpu.SemaphoreType.DMA,
        ],
    )
    def label_mean(labels_hbm, table_hbm, out_hbm,
                   lab_v, rows0, rows1, out_v, sem0, sem1):
        wid = lax.axis_index("s") * NUM_CORES + lax.axis_index("c")
        base = wid * PER_W
        rows = (rows0, rows1)
        sems = (sem0, sem1)

        pltpu.sync_copy(labels_hbm.at[pl.ds(base * SEQ, PER_W * SEQ)], lab_v)

        def fire(le, p):
            pltpu.async_copy(
                table_hbm.at[lab_v.at[pl.ds(le * SEQ, C0)]],
                rows[p].at[pl.ds(0, C0)],
                sems[p],
            )
            pltpu.async_copy(
                table_hbm.at[lab_v.at[pl.ds(le * SEQ + C0, C1)]],
                rows[p].at[pl.ds(C0, C1)],
                sems[p],
            )

        def drain(p):
            pltpu.make_async_copy(
                table_hbm.at[pl.ds(0, SEQ)], rows[p], sems[p]
            ).wait()

        scale = jnp.float32(1.0 / SEQ)
        zero = jnp.zeros((LANES,), jnp.float32)
        fire(0, 0)

        def group_body(g, carry):
            for e in range(GROUP):
                le = g * GROUP + e
                p = e % 2
                nxt = le + 1

                @pl.when(nxt < PER_W)
                def _():
                    fire(nxt, (e + 1) % 2)

                drain(p)
                buf = rows[p]

                def red(r, accs):
                    a = list(accs)
                    for u in range(UNROLL):
                        s = (u % 2) * VREGS
                        for k in range(VREGS):
                            a[s + k] = a[s + k] + buf[
                                r * UNROLL + u, pl.ds(k * LANES, LANES)
                            ]
                    return tuple(a)

                accs = lax.fori_loop(0, SEQ // UNROLL, red, (zero,) * (2 * VREGS))
                for k in range(VREGS):
                    out_v[e, pl.ds(k * LANES, LANES)] = (
                        accs[k] + accs[VREGS + k]
                    ) * scale
            pltpu.sync_copy(out_v, out_hbm.at[pl.ds(base + g * GROUP, GROUP)])
            return carry

        lax.fori_loop(0, PER_W // GROUP, group_body, 0)

    return label_mean


_label_mean = _make_kernel()


@jax.jit
def kernel(labels, table):
    labels_flat = labels.astype(jnp.int32).reshape(BATCH * SEQ)
    return _label_mean(labels_flat, table)
